# Initial kernel scaffold; baseline (speedup 1.0000x reference)
import jax
import jax.numpy as jnp
from jax.experimental import pallas as pl


def kernel(x, e, edge_index, W_xe, b_xe, W_ee, b_ee, U, Ub, V, Vb, A, Ab, Bm, Bb, Cm, Cb, gnx, bnx, gne, bne, W_outx, b_outx, W_oute, b_oute):
    raise NotImplementedError("write your pallas kernel here")


# speedup vs baseline: 1.0475x; 1.0475x over previous
import functools

import jax
import jax.numpy as jnp
from jax.experimental import pallas as pl
from jax.experimental.pallas import tpu as pltpu

_BN = 1000
_BE = 2000


def _ln_act(v, gamma, beta):
    mu = jnp.mean(v, axis=-1, keepdims=True)
    var = jnp.mean((v - mu) ** 2, axis=-1, keepdims=True)
    return jax.nn.relu(gamma * (v - mu) * jax.lax.rsqrt(var + 1e-5) + beta)


def _mm_body(h_ref, w_ref, b_ref, out_ref):
    out_ref[...] = (
        jnp.dot(h_ref[...], w_ref[...], preferred_element_type=jnp.float32)
        + b_ref[...]
    )


def _mm(h, w, b, bn):
    n, d = h.shape
    m = w.shape[1]
    return pl.pallas_call(
        _mm_body,
        grid=(n // bn,),
        in_specs=[
            pl.BlockSpec((bn, d), lambda i: (i, 0)),
            pl.BlockSpec((d, m), lambda i: (0, 0)),
            pl.BlockSpec((1, m), lambda i: (0, 0)),
        ],
        out_specs=pl.BlockSpec((bn, m), lambda i: (i, 0)),
        out_shape=jax.ShapeDtypeStruct((n, m), jnp.float32),
    )(h, w, b.reshape(1, m))


def _edge_body(g_ref, s_ref, vhs_ref, a_ref, ab_ref, gne_ref, bne_ref,
               msg_ref, gnew_ref):
    g = g_ref[...]
    e_new = (
        jnp.dot(g, a_ref[...], preferred_element_type=jnp.float32)
        + ab_ref[...]
        + s_ref[...]
    )
    msg_ref[...] = jax.nn.sigmoid(e_new) * vhs_ref[...]
    gnew_ref[...] = g + _ln_act(e_new, gne_ref[...], bne_ref[...])


def _edge_fused(g, s, vhs, a, ab, gne_i, bne_i):
    ecount, d = g.shape
    vec = pl.BlockSpec((1, d), lambda i: (0, 0))
    blk = pl.BlockSpec((_BE, d), lambda i: (i, 0))
    return pl.pallas_call(
        _edge_body,
        grid=(ecount // _BE,),
        in_specs=[blk, blk, blk, pl.BlockSpec((d, d), lambda i: (0, 0)),
                  vec, vec, vec],
        out_specs=[blk, blk],
        out_shape=[
            jax.ShapeDtypeStruct((ecount, d), jnp.float32),
            jax.ShapeDtypeStruct((ecount, d), jnp.float32),
        ],
    )(g, s, vhs, a, ab.reshape(1, d), gne_i.reshape(1, d),
      bne_i.reshape(1, d))


def _hupd_body(h_ref, uh_ref, agg_ref, gnx_ref, bnx_ref, out_ref):
    out_ref[...] = h_ref[...] + _ln_act(
        uh_ref[...] + agg_ref[...], gnx_ref[...], bnx_ref[...])


def _h_update(h, uh, agg, gnx_i, bnx_i):
    n, d = h.shape
    vec = pl.BlockSpec((1, d), lambda i: (0, 0))
    blk = pl.BlockSpec((_BN, d), lambda i: (i, 0))
    return pl.pallas_call(
        _hupd_body,
        grid=(n // _BN,),
        in_specs=[blk, blk, blk, vec, vec],
        out_specs=blk,
        out_shape=jax.ShapeDtypeStruct((n, d), jnp.float32),
    )(h, uh, agg, gnx_i.reshape(1, d), bnx_i.reshape(1, d))


def kernel(x, e, edge_index, W_xe, b_xe, W_ee, b_ee, U, Ub, V, Vb, A, Ab,
           Bm, Bb, Cm, Cb, gnx, bnx, gne, bne, W_outx, b_outx, W_oute,
           b_oute):
    n = x.shape[0]
    ecount = e.shape[0]
    d = W_xe.shape[1]
    layers = U.shape[0]
    src = edge_index[0]
    dst = edge_index[1]

    h = x @ W_xe + b_xe
    g = e * W_ee[0] + b_ee

    for i in range(layers):
        wcat = jnp.concatenate([U[i], V[i], Bm[i], Cm[i]], axis=1)
        bcat = jnp.concatenate([Ub[i], Vb[i], Bb[i], Cb[i]], axis=0)
        hw = _mm(h, wcat, bcat, _BN)
        uh, vh, bh, ch = jnp.split(hw, 4, axis=1)
        s = bh[src] + ch[dst]
        vhs = vh[src]
        msg, gnew = _edge_fused(g, s, vhs, A[i], Ab[i], gne[i], bne[i])
        agg = jnp.zeros((n, d), jnp.float32).at[dst].add(msg)
        h = _h_update(h, uh, agg, gnx[i], bnx[i])
        g = gnew

    x_out = _mm(h, W_outx, b_outx, _BN)
    e_out = _mm(g, W_oute, b_oute, _BE)
    return (x_out, e_out)


# speedup vs baseline: 2.8510x; 2.7218x over previous
import functools

import jax
import jax.numpy as jnp
from jax import lax
from jax.experimental import pallas as pl
from jax.experimental.pallas import tpu as pltpu
from jax.experimental.pallas import tpu_sc as plsc

_BN = 1000
_BE = 2000
_NC = 2
_NS = 16
_NW = _NC * _NS
_KC = 80


def _ln_act(v, gamma, beta):
    mu = jnp.mean(v, axis=-1, keepdims=True)
    var = jnp.mean((v - mu) ** 2, axis=-1, keepdims=True)
    return jax.nn.relu(gamma * (v - mu) * jax.lax.rsqrt(var + 1e-5) + beta)


def _mm_body(h_ref, w_ref, b_ref, out_ref):
    out_ref[...] = (
        jnp.dot(h_ref[...], w_ref[...], preferred_element_type=jnp.float32)
        + b_ref[...]
    )


def _mm(h, w, b, bn):
    n, d = h.shape
    m = w.shape[1]
    return pl.pallas_call(
        _mm_body,
        grid=(n // bn,),
        in_specs=[
            pl.BlockSpec((bn, d), lambda i: (i, 0)),
            pl.BlockSpec((d, m), lambda i: (0, 0)),
            pl.BlockSpec((1, m), lambda i: (0, 0)),
        ],
        out_specs=pl.BlockSpec((bn, m), lambda i: (i, 0)),
        out_shape=jax.ShapeDtypeStruct((n, m), jnp.float32),
    )(h, w, b.reshape(1, m))


def _mm4_body(h_ref, w_ref, b_ref, o0, o1, o2, o3):
    hw = (
        jnp.dot(h_ref[...], w_ref[...], preferred_element_type=jnp.float32)
        + b_ref[...]
    )
    d = o0.shape[1]
    o0[...] = hw[:, :d]
    o1[...] = hw[:, d:2 * d]
    o2[...] = hw[:, 2 * d:3 * d]
    o3[...] = hw[:, 3 * d:]


def _mm4(h, wcat, bcat):
    n, d = h.shape
    blk = pl.BlockSpec((_BN, d), lambda i: (i, 0))
    out = jax.ShapeDtypeStruct((n, d), jnp.float32)
    return pl.pallas_call(
        _mm4_body,
        grid=(n // _BN,),
        in_specs=[
            blk,
            pl.BlockSpec((d, 4 * d), lambda i: (0, 0)),
            pl.BlockSpec((1, 4 * d), lambda i: (0, 0)),
        ],
        out_specs=[blk, blk, blk, blk],
        out_shape=[out, out, out, out],
    )(h, wcat, bcat.reshape(1, 4 * d))


def _edge_body(g_ref, s_ref, a_ref, ab_ref, gne_ref, bne_ref,
               gates_ref, gnew_ref):
    g = g_ref[...]
    e_new = (
        jnp.dot(g, a_ref[...], preferred_element_type=jnp.float32)
        + ab_ref[...]
        + s_ref[...]
    )
    gates_ref[...] = jax.nn.sigmoid(e_new)
    gnew_ref[...] = g + _ln_act(e_new, gne_ref[...], bne_ref[...])


def _edge_fused(g, s, a, ab, gne_i, bne_i):
    ecount, d = g.shape
    vec = pl.BlockSpec((1, d), lambda i: (0, 0))
    blk = pl.BlockSpec((_BE, d), lambda i: (i, 0))
    out = jax.ShapeDtypeStruct((ecount, d), jnp.float32)
    return pl.pallas_call(
        _edge_body,
        grid=(ecount // _BE,),
        in_specs=[blk, blk, pl.BlockSpec((d, d), lambda i: (0, 0)),
                  vec, vec, vec],
        out_specs=[blk, blk],
        out_shape=[out, out],
    )(g, s, a, ab.reshape(1, d), gne_i.reshape(1, d), bne_i.reshape(1, d))


def _hupd_body(h_ref, uh_ref, a0_ref, a1_ref, gnx_ref, bnx_ref, out_ref):
    out_ref[...] = h_ref[...] + _ln_act(
        uh_ref[...] + a0_ref[...] + a1_ref[...], gnx_ref[...], bnx_ref[...])


def _h_update(h, uh, agg0, agg1, gnx_i, bnx_i):
    n, d = h.shape
    vec = pl.BlockSpec((1, d), lambda i: (0, 0))
    blk = pl.BlockSpec((_BN, d), lambda i: (i, 0))
    return pl.pallas_call(
        _hupd_body,
        grid=(n // _BN,),
        in_specs=[blk, blk, blk, blk, vec, vec],
        out_specs=blk,
        out_shape=jax.ShapeDtypeStruct((n, d), jnp.float32),
    )(h, uh, agg0, agg1, gnx_i.reshape(1, d), bnx_i.reshape(1, d))


def _scg_body(bh, ch, src, dst, s_out,
              idx_s, idx_d, buf_b, buf_c, sem_b, sem_c):
    c = lax.axis_index("c")
    s = lax.axis_index("s")
    wid = s * _NC + c
    ecount = src.shape[0]
    per_w = ecount // _NW
    chunks = per_w // _KC
    base = wid * per_w

    def chunk(ci, carry):
        off = base + ci * _KC
        pltpu.sync_copy(src.at[pl.ds(off, _KC)], idx_s)
        pltpu.sync_copy(dst.at[pl.ds(off, _KC)], idx_d)
        cb = pltpu.async_copy(bh.at[idx_s], buf_b, sem_b)
        cc = pltpu.async_copy(ch.at[idx_d], buf_c, sem_c)
        cb.wait()
        cc.wait()

        def row(r, rc):
            for j in range(8):
                sl = (r, pl.ds(j * 16, 16))
                buf_b[sl] = buf_b[sl] + buf_c[sl]
            return rc

        lax.fori_loop(0, _KC, row, 0)
        pltpu.sync_copy(buf_b, s_out.at[pl.ds(off, _KC)])
        return carry

    lax.fori_loop(0, chunks, chunk, 0)


def _sc_gather_s(bh, ch, src, dst):
    ecount = src.shape[0]
    d = bh.shape[1]
    return pl.kernel(
        _scg_body,
        out_type=jax.ShapeDtypeStruct((ecount, d), jnp.float32),
        mesh=plsc.VectorSubcoreMesh(core_axis_name="c", subcore_axis_name="s"),
        scratch_types=[
            pltpu.VMEM((_KC,), jnp.int32),
            pltpu.VMEM((_KC,), jnp.int32),
            pltpu.VMEM((_KC, d), jnp.float32),
            pltpu.VMEM((_KC, d), jnp.float32),
            pltpu.SemaphoreType.DMA,
            pltpu.SemaphoreType.DMA,
        ],
    )(bh, ch, src, dst)


_NSTAGE = 200


def _scs_body(gates, vh, src, dst, agg0, agg1,
              idx_s, idx_d, buf_g, buf_v, stage, acc, sem_v):
    c = lax.axis_index("c")
    s = lax.axis_index("s")
    wid = s * _NC + c
    n = vh.shape[0]
    ecount = gates.shape[0]
    per_w = ecount // _NW
    chunks = per_w // _KC
    base = wid * per_w
    nchunks = n // _NSTAGE

    zero = jnp.zeros((16,), jnp.float32)

    def zrow(r, carry):
        for j in range(8):
            stage[r, pl.ds(j * 16, 16)] = zero
        return carry

    lax.fori_loop(0, _NSTAGE, zrow, 0)

    def zcopy(t, carry):
        ct = t * _NS + s

        @pl.when(ct < nchunks)
        def _():
            pltpu.sync_copy(stage, acc.at[pl.ds(ct * _NSTAGE, _NSTAGE)])

        return carry

    lax.fori_loop(0, pl.cdiv(nchunks, _NS), zcopy, 0)
    plsc.subcore_barrier()

    def chunk(ci, carry):
        off = base + ci * _KC
        pltpu.sync_copy(src.at[pl.ds(off, _KC)], idx_s)
        pltpu.sync_copy(dst.at[pl.ds(off, _KC)], idx_d)
        cv = pltpu.async_copy(vh.at[idx_s], buf_v, sem_v)
        pltpu.sync_copy(gates.at[pl.ds(off, _KC)], buf_g)
        cv.wait()

        def row(r, rc):
            for j in range(8):
                sl = (r, pl.ds(j * 16, 16))
                buf_v[sl] = buf_v[sl] * buf_g[sl]
            return rc

        lax.fori_loop(0, _KC, row, 0)
        pltpu.sync_copy(buf_v, acc.at[idx_d], add=True)
        return carry

    lax.fori_loop(0, chunks, chunk, 0)
    plsc.subcore_barrier()

    def flush(t, carry):
        ct = t * _NS + s

        @pl.when(ct < nchunks)
        def _():
            r0 = ct * _NSTAGE
            pltpu.sync_copy(acc.at[pl.ds(r0, _NSTAGE)], stage)

            @pl.when(c == 0)
            def _():
                pltpu.sync_copy(stage, agg0.at[pl.ds(r0, _NSTAGE)])

            @pl.when(c == 1)
            def _():
                pltpu.sync_copy(stage, agg1.at[pl.ds(r0, _NSTAGE)])

        return carry

    lax.fori_loop(0, pl.cdiv(nchunks, _NS), flush, 0)


def _sc_scatter(gates, vh, src, dst):
    n, d = vh.shape
    out = jax.ShapeDtypeStruct((n, d), jnp.float32)
    return pl.kernel(
        _scs_body,
        out_type=[out, out],
        mesh=plsc.VectorSubcoreMesh(core_axis_name="c", subcore_axis_name="s"),
        scratch_types=[
            pltpu.VMEM((_KC,), jnp.int32),
            pltpu.VMEM((_KC,), jnp.int32),
            pltpu.VMEM((_KC, d), jnp.float32),
            pltpu.VMEM((_KC, d), jnp.float32),
            pltpu.VMEM((_NSTAGE, d), jnp.float32),
            pltpu.VMEM_SHARED((n, d), jnp.float32),
            pltpu.SemaphoreType.DMA,
        ],
    )(gates, vh, src, dst)


def kernel(x, e, edge_index, W_xe, b_xe, W_ee, b_ee, U, Ub, V, Vb, A, Ab,
           Bm, Bb, Cm, Cb, gnx, bnx, gne, bne, W_outx, b_outx, W_oute,
           b_oute):
    d = W_xe.shape[1]
    layers = U.shape[0]
    src = edge_index[0]
    dst = edge_index[1]

    h = x @ W_xe + b_xe
    g = e * W_ee[0] + b_ee

    for i in range(layers):
        wcat = jnp.concatenate([U[i], V[i], Bm[i], Cm[i]], axis=1)
        bcat = jnp.concatenate([Ub[i], Vb[i], Bb[i], Cb[i]], axis=0)
        uh, vh, bh, ch = _mm4(h, wcat, bcat)
        s = _sc_gather_s(bh, ch, src, dst)
        gates, gnew = _edge_fused(g, s, A[i], Ab[i], gne[i], bne[i])
        agg0, agg1 = _sc_scatter(gates, vh, src, dst)
        h = _h_update(h, uh, agg0, agg1, gnx[i], bnx[i])
        g = gnew

    x_out = _mm(h, W_outx, b_outx, _BN)
    e_out = _mm(g, W_oute, b_oute, _BE)
    return (x_out, e_out)


# speedup vs baseline: 4.3130x; 1.5128x over previous
import functools

import jax
import jax.numpy as jnp
from jax import lax
from jax.experimental import pallas as pl
from jax.experimental.pallas import tpu as pltpu
from jax.experimental.pallas import tpu_sc as plsc

_BN = 1000
_BE = 2000
_NC = 2
_NS = 16
_NW = _NC * _NS
_KC = 80


def _ln_act(v, gamma, beta):
    mu = jnp.mean(v, axis=-1, keepdims=True)
    var = jnp.mean((v - mu) ** 2, axis=-1, keepdims=True)
    return jax.nn.relu(gamma * (v - mu) * jax.lax.rsqrt(var + 1e-5) + beta)


def _mm_body(h_ref, w_ref, b_ref, out_ref):
    out_ref[...] = (
        jnp.dot(h_ref[...], w_ref[...], preferred_element_type=jnp.float32)
        + b_ref[...]
    )


def _mm(h, w, b, bn):
    n, d = h.shape
    m = w.shape[1]
    return pl.pallas_call(
        _mm_body,
        grid=(n // bn,),
        in_specs=[
            pl.BlockSpec((bn, d), lambda i: (i, 0)),
            pl.BlockSpec((d, m), lambda i: (0, 0)),
            pl.BlockSpec((1, m), lambda i: (0, 0)),
        ],
        out_specs=pl.BlockSpec((bn, m), lambda i: (i, 0)),
        out_shape=jax.ShapeDtypeStruct((n, m), jnp.float32),
    )(h, w, b.reshape(1, m))


def _mm4_body(h_ref, w_ref, b_ref, o0, o1, o2, o3):
    hw = (
        jnp.dot(h_ref[...], w_ref[...], preferred_element_type=jnp.float32)
        + b_ref[...]
    )
    d = o0.shape[1]
    o0[...] = hw[:, :d]
    o1[...] = hw[:, d:2 * d]
    o2[...] = hw[:, 2 * d:3 * d]
    o3[...] = hw[:, 3 * d:]


def _mm4(h, wcat, bcat):
    n, d = h.shape
    blk = pl.BlockSpec((_BN, d), lambda i: (i, 0))
    out = jax.ShapeDtypeStruct((n, d), jnp.float32)
    return pl.pallas_call(
        _mm4_body,
        grid=(n // _BN,),
        in_specs=[
            blk,
            pl.BlockSpec((d, 4 * d), lambda i: (0, 0)),
            pl.BlockSpec((1, 4 * d), lambda i: (0, 0)),
        ],
        out_specs=[blk, blk, blk, blk],
        out_shape=[out, out, out, out],
    )(h, wcat, bcat.reshape(1, 4 * d))


def _edge_body(g_ref, s_ref, a_ref, ab_ref, gne_ref, bne_ref,
               gates_ref, gnew_ref):
    g = g_ref[...]
    e_new = (
        jnp.dot(g, a_ref[...], preferred_element_type=jnp.float32)
        + ab_ref[...]
        + s_ref[...]
    )
    gates_ref[...] = jax.nn.sigmoid(e_new)
    gnew_ref[...] = g + _ln_act(e_new, gne_ref[...], bne_ref[...])


def _edge_fused(g, s, a, ab, gne_i, bne_i):
    ecount, d = g.shape
    vec = pl.BlockSpec((1, d), lambda i: (0, 0))
    blk = pl.BlockSpec((_BE, d), lambda i: (i, 0))
    out = jax.ShapeDtypeStruct((ecount, d), jnp.float32)
    return pl.pallas_call(
        _edge_body,
        grid=(ecount // _BE,),
        in_specs=[blk, blk, pl.BlockSpec((d, d), lambda i: (0, 0)),
                  vec, vec, vec],
        out_specs=[blk, blk],
        out_shape=[out, out],
    )(g, s, a, ab.reshape(1, d), gne_i.reshape(1, d), bne_i.reshape(1, d))


def _hupd_body(h_ref, uh_ref, a0_ref, a1_ref, gnx_ref, bnx_ref, out_ref):
    out_ref[...] = h_ref[...] + _ln_act(
        uh_ref[...] + a0_ref[...] + a1_ref[...], gnx_ref[...], bnx_ref[...])


def _h_update(h, uh, agg0, agg1, gnx_i, bnx_i):
    n, d = h.shape
    vec = pl.BlockSpec((1, d), lambda i: (0, 0))
    blk = pl.BlockSpec((_BN, d), lambda i: (i, 0))
    return pl.pallas_call(
        _hupd_body,
        grid=(n // _BN,),
        in_specs=[blk, blk, blk, blk, vec, vec],
        out_specs=blk,
        out_shape=jax.ShapeDtypeStruct((n, d), jnp.float32),
    )(h, uh, agg0, agg1, gnx_i.reshape(1, d), bnx_i.reshape(1, d))


def _scg_body(bh, ch, src3, dst3, s_out,
              idxs, idxd, bb0, bc0, bb1, bc1,
              semb0, semc0, semb1, semc1, semo0, semo1):
    c = lax.axis_index("c")
    s = lax.axis_index("s")
    wid = s * _NC + c
    chunks = src3.shape[1]
    base = wid * chunks * _KC
    pltpu.sync_copy(src3.at[wid], idxs)
    pltpu.sync_copy(dst3.at[wid], idxd)

    def start(ci, bb, bc, semb, semc):
        pltpu.async_copy(bh.at[idxs.at[ci]], bb, semb)
        pltpu.async_copy(ch.at[idxd.at[ci]], bc, semc)

    def work(ci, bb, bc, semb, semc, semo):
        pltpu.make_async_copy(bh.at[pl.ds(0, _KC)], bb, semb).wait()
        pltpu.make_async_copy(ch.at[pl.ds(0, _KC)], bc, semc).wait()

        def row(r, rc):
            for j in range(8):
                sl = (r, pl.ds(j * 16, 16))
                bb[sl] = bb[sl] + bc[sl]
            return rc

        lax.fori_loop(0, _KC, row, 0)
        pltpu.async_copy(bb, s_out.at[pl.ds(base + ci * _KC, _KC)], semo)

    def wait_store(bb, semo):
        pltpu.make_async_copy(bb, s_out.at[pl.ds(base, _KC)], semo).wait()

    start(0, bb0, bc0, semb0, semc0)

    def step(t, carry):
        @pl.when(t % 2 == 0)
        def _():
            @pl.when(t > 0)
            def _():
                wait_store(bb1, semo1)

            @pl.when(t + 1 < chunks)
            def _():
                start(t + 1, bb1, bc1, semb1, semc1)

            work(t, bb0, bc0, semb0, semc0, semo0)

        @pl.when(t % 2 == 1)
        def _():
            wait_store(bb0, semo0)

            @pl.when(t + 1 < chunks)
            def _():
                start(t + 1, bb0, bc0, semb0, semc0)

            work(t, bb1, bc1, semb1, semc1, semo1)

        return carry

    lax.fori_loop(0, chunks, step, 0)
    wait_store(bb0, semo0)


def _sc_gather_s(bh, ch, src3, dst3):
    ecount = src3.size
    d = bh.shape[1]
    chunks = src3.shape[1]
    assert chunks % 2 == 1
    buf = pltpu.VMEM((_KC, d), jnp.float32)
    idx = pltpu.VMEM((chunks, _KC), jnp.int32)
    return pl.kernel(
        _scg_body,
        out_type=jax.ShapeDtypeStruct((ecount, d), jnp.float32),
        mesh=plsc.VectorSubcoreMesh(core_axis_name="c", subcore_axis_name="s"),
        scratch_types=[idx, idx, buf, buf, buf, buf]
        + [pltpu.SemaphoreType.DMA] * 6,
    )(bh, ch, src3, dst3)


def _scs_body(gates, vh, src, dst, agg0, agg1,
              is0, id0, is1, id1, bg0, bv0, bg1, bv1, acc,
              semg0, semv0, semg1, semv1, sems0, sems1):
    c = lax.axis_index("c")
    s = lax.axis_index("s")
    wid = s * _NC + c
    n = vh.shape[0]
    per_w = gates.shape[0] // _NW
    chunks = per_w // _KC
    base = wid * per_w
    nrch = n // _KC

    zero = jnp.zeros((16,), jnp.float32)

    def zrow(r, carry):
        for j in range(8):
            bv0[r, pl.ds(j * 16, 16)] = zero
        return carry

    lax.fori_loop(0, _KC, zrow, 0)

    def zcopy(t, carry):
        ct = t * _NS + s

        @pl.when(ct < nrch)
        def _():
            pltpu.sync_copy(bv0, acc.at[pl.ds(ct * _KC, _KC)])

        return carry

    lax.fori_loop(0, pl.cdiv(nrch, _NS), zcopy, 0)
    plsc.subcore_barrier()

    def start(ci, isb, idb, bg, bv, semg, semv):
        off = base + ci * _KC
        pltpu.sync_copy(src.at[pl.ds(off, _KC)], isb)
        pltpu.sync_copy(dst.at[pl.ds(off, _KC)], idb)
        pltpu.async_copy(vh.at[isb], bv, semv)
        pltpu.async_copy(gates.at[pl.ds(off, _KC)], bg, semg)

    def work(ci, idb, bg, bv, semg, semv, semsc):
        pltpu.make_async_copy(vh.at[pl.ds(0, _KC)], bv, semv).wait()
        pltpu.make_async_copy(vh.at[pl.ds(0, _KC)], bg, semg).wait()

        def row(r, rc):
            for j in range(8):
                sl = (r, pl.ds(j * 16, 16))
                bv[sl] = bv[sl] * bg[sl]
            return rc

        lax.fori_loop(0, _KC, row, 0)
        pltpu.async_copy(bv, acc.at[idb], semsc, add=True)

    def wait_scat(bv, semsc):
        pltpu.make_async_copy(bv, acc.at[pl.ds(0, _KC)], semsc).wait()

    start(0, is0, id0, bg0, bv0, semg0, semv0)

    def step(t, carry):
        @pl.when(t % 2 == 0)
        def _():
            @pl.when(t > 0)
            def _():
                wait_scat(bv1, sems1)

            @pl.when(t + 1 < chunks)
            def _():
                start(t + 1, is1, id1, bg1, bv1, semg1, semv1)

            work(t, id0, bg0, bv0, semg0, semv0, sems0)

        @pl.when(t % 2 == 1)
        def _():
            wait_scat(bv0, sems0)

            @pl.when(t + 1 < chunks)
            def _():
                start(t + 1, is0, id0, bg0, bv0, semg0, semv0)

            work(t, id1, bg1, bv1, semg1, semv1, sems1)

        return carry

    lax.fori_loop(0, chunks, step, 0)
    wait_scat(bv0, sems0)
    plsc.subcore_barrier()

    def flush(t, carry):
        ct = t * _NS + s

        @pl.when(ct < nrch)
        def _():
            r0 = ct * _KC
            pltpu.sync_copy(acc.at[pl.ds(r0, _KC)], bv0)

            @pl.when(c == 0)
            def _():
                pltpu.sync_copy(bv0, agg0.at[pl.ds(r0, _KC)])

            @pl.when(c == 1)
            def _():
                pltpu.sync_copy(bv0, agg1.at[pl.ds(r0, _KC)])

        return carry

    lax.fori_loop(0, pl.cdiv(nrch, _NS), flush, 0)


def _sc_scatter(gates, vh, src, dst):
    n, d = vh.shape
    assert (gates.shape[0] // _NW // _KC) % 2 == 1
    out = jax.ShapeDtypeStruct((n, d), jnp.float32)
    buf = pltpu.VMEM((_KC, d), jnp.float32)
    idx = pltpu.VMEM((_KC,), jnp.int32)
    return pl.kernel(
        _scs_body,
        out_type=[out, out],
        mesh=plsc.VectorSubcoreMesh(core_axis_name="c", subcore_axis_name="s"),
        scratch_types=[idx, idx, idx, idx, buf, buf, buf, buf,
                       pltpu.VMEM_SHARED((n, d), jnp.float32)]
        + [pltpu.SemaphoreType.DMA] * 6,
    )(gates, vh, src, dst)


def kernel(x, e, edge_index, W_xe, b_xe, W_ee, b_ee, U, Ub, V, Vb, A, Ab,
           Bm, Bb, Cm, Cb, gnx, bnx, gne, bne, W_outx, b_outx, W_oute,
           b_oute):
    d = W_xe.shape[1]
    layers = U.shape[0]
    ecount = e.shape[0]
    chunks = ecount // _NW // _KC
    src1 = edge_index[0]
    dst1 = edge_index[1]
    src3 = src1.reshape(_NW, chunks, _KC)
    dst3 = dst1.reshape(_NW, chunks, _KC)

    h = x @ W_xe + b_xe
    g = e * W_ee[0] + b_ee

    for i in range(layers):
        wcat = jnp.concatenate([U[i], V[i], Bm[i], Cm[i]], axis=1)
        bcat = jnp.concatenate([Ub[i], Vb[i], Bb[i], Cb[i]], axis=0)
        uh, vh, bh, ch = _mm4(h, wcat, bcat)
        s = _sc_gather_s(bh, ch, src3, dst3)
        gates, gnew = _edge_fused(g, s, A[i], Ab[i], gne[i], bne[i])
        agg0, agg1 = _sc_scatter(gates, vh, src1, dst1)
        h = _h_update(h, uh, agg0, agg1, gnx[i], bnx[i])
        g = gnew

    x_out = _mm(h, W_outx, b_outx, _BN)
    e_out = _mm(g, W_oute, b_oute, _BE)
    return (x_out, e_out)


# speedup vs baseline: 4.3364x; 1.0054x over previous
import functools

import jax
import jax.numpy as jnp
from jax import lax
from jax.experimental import pallas as pl
from jax.experimental.pallas import tpu as pltpu
from jax.experimental.pallas import tpu_sc as plsc

_BN = 1000
_BE = 2000
_NC = 2
_NS = 16
_NW = _NC * _NS
_KC = 80


def _ln_act(v, gamma, beta):
    mu = jnp.mean(v, axis=-1, keepdims=True)
    var = jnp.mean((v - mu) ** 2, axis=-1, keepdims=True)
    return jax.nn.relu(gamma * (v - mu) * jax.lax.rsqrt(var + 1e-5) + beta)


def _mm_body(h_ref, w_ref, b_ref, out_ref):
    out_ref[...] = (
        jnp.dot(h_ref[...], w_ref[...], preferred_element_type=jnp.float32)
        + b_ref[...]
    )


def _mm(h, w, b, bn):
    n, d = h.shape
    m = w.shape[1]
    return pl.pallas_call(
        _mm_body,
        grid=(n // bn,),
        in_specs=[
            pl.BlockSpec((bn, d), lambda i: (i, 0)),
            pl.BlockSpec((d, m), lambda i: (0, 0)),
            pl.BlockSpec((1, m), lambda i: (0, 0)),
        ],
        out_specs=pl.BlockSpec((bn, m), lambda i: (i, 0)),
        out_shape=jax.ShapeDtypeStruct((n, m), jnp.float32),
    )(h, w, b.reshape(1, m))


def _mm4_body(h_ref, w_ref, b_ref, o0, o1, o2, o3):
    hw = (
        jnp.dot(h_ref[...], w_ref[...], preferred_element_type=jnp.float32)
        + b_ref[...]
    )
    d = o0.shape[1]
    o0[...] = hw[:, :d]
    o1[...] = hw[:, d:2 * d]
    o2[...] = hw[:, 2 * d:3 * d]
    o3[...] = hw[:, 3 * d:]


def _mm4(h, wcat, bcat):
    n, d = h.shape
    blk = pl.BlockSpec((_BN, d), lambda i: (i, 0))
    out = jax.ShapeDtypeStruct((n, d), jnp.float32)
    return pl.pallas_call(
        _mm4_body,
        grid=(n // _BN,),
        in_specs=[
            blk,
            pl.BlockSpec((d, 4 * d), lambda i: (0, 0)),
            pl.BlockSpec((1, 4 * d), lambda i: (0, 0)),
        ],
        out_specs=[blk, blk, blk, blk],
        out_shape=[out, out, out, out],
    )(h, wcat, bcat.reshape(1, 4 * d))


def _edge_body(g_ref, s_ref, a_ref, ab_ref, gne_ref, bne_ref,
               gates_ref, gnew_ref):
    g = g_ref[...]
    e_new = (
        jnp.dot(g, a_ref[...], preferred_element_type=jnp.float32)
        + ab_ref[...]
        + s_ref[...]
    )
    gates_ref[...] = jax.nn.sigmoid(e_new)
    gnew_ref[...] = g + _ln_act(e_new, gne_ref[...], bne_ref[...])


def _edge_fused(g, s, a, ab, gne_i, bne_i):
    ecount, d = g.shape
    vec = pl.BlockSpec((1, d), lambda i: (0, 0))
    blk = pl.BlockSpec((_BE, d), lambda i: (i, 0))
    out = jax.ShapeDtypeStruct((ecount, d), jnp.float32)
    return pl.pallas_call(
        _edge_body,
        grid=(ecount // _BE,),
        in_specs=[blk, blk, pl.BlockSpec((d, d), lambda i: (0, 0)),
                  vec, vec, vec],
        out_specs=[blk, blk],
        out_shape=[out, out],
    )(g, s, a, ab.reshape(1, d), gne_i.reshape(1, d), bne_i.reshape(1, d))


def _edge_first_body(e_ref, wee_ref, bee_ref, s_ref, a_ref, ab_ref,
                     gne_ref, bne_ref, gates_ref, gnew_ref):
    g = e_ref[...] * wee_ref[...] + bee_ref[...]
    e_new = (
        jnp.dot(g, a_ref[...], preferred_element_type=jnp.float32)
        + ab_ref[...]
        + s_ref[...]
    )
    gates_ref[...] = jax.nn.sigmoid(e_new)
    gnew_ref[...] = g + _ln_act(e_new, gne_ref[...], bne_ref[...])


def _edge_fused_first(e, wee, bee, s, a, ab, gne_i, bne_i):
    ecount, d = s.shape
    vec = pl.BlockSpec((1, d), lambda i: (0, 0))
    blk = pl.BlockSpec((_BE, d), lambda i: (i, 0))
    out = jax.ShapeDtypeStruct((ecount, d), jnp.float32)
    return pl.pallas_call(
        _edge_first_body,
        grid=(ecount // _BE,),
        in_specs=[pl.BlockSpec((_BE, 1), lambda i: (i, 0)), vec, vec, blk,
                  pl.BlockSpec((d, d), lambda i: (0, 0)), vec, vec, vec],
        out_specs=[blk, blk],
        out_shape=[out, out],
    )(e, wee.reshape(1, d), bee.reshape(1, d), s, a, ab.reshape(1, d),
      gne_i.reshape(1, d), bne_i.reshape(1, d))


def _edge_last_body(g_ref, s_ref, a_ref, ab_ref, gne_ref, bne_ref,
                    wo_ref, bo_ref, gates_ref, eout_ref):
    g = g_ref[...]
    e_new = (
        jnp.dot(g, a_ref[...], preferred_element_type=jnp.float32)
        + ab_ref[...]
        + s_ref[...]
    )
    gates_ref[...] = jax.nn.sigmoid(e_new)
    gnew = g + _ln_act(e_new, gne_ref[...], bne_ref[...])
    eout_ref[...] = (
        jnp.dot(gnew, wo_ref[...], preferred_element_type=jnp.float32)
        + bo_ref[...]
    )


def _edge_fused_last(g, s, a, ab, gne_i, bne_i, woute, boute):
    ecount, d = g.shape
    m = woute.shape[1]
    vec = pl.BlockSpec((1, d), lambda i: (0, 0))
    blk = pl.BlockSpec((_BE, d), lambda i: (i, 0))
    return pl.pallas_call(
        _edge_last_body,
        grid=(ecount // _BE,),
        in_specs=[blk, blk, pl.BlockSpec((d, d), lambda i: (0, 0)),
                  vec, vec, vec,
                  pl.BlockSpec((d, m), lambda i: (0, 0)),
                  pl.BlockSpec((1, m), lambda i: (0, 0))],
        out_specs=[blk, pl.BlockSpec((_BE, m), lambda i: (i, 0))],
        out_shape=[jax.ShapeDtypeStruct((ecount, d), jnp.float32),
                   jax.ShapeDtypeStruct((ecount, m), jnp.float32)],
    )(g, s, a, ab.reshape(1, d), gne_i.reshape(1, d), bne_i.reshape(1, d),
      woute, boute.reshape(1, m))


def _hupd_last_body(h_ref, uh_ref, a0_ref, a1_ref, gnx_ref, bnx_ref,
                    wo_ref, bo_ref, xout_ref):
    hnew = h_ref[...] + _ln_act(
        uh_ref[...] + a0_ref[...] + a1_ref[...], gnx_ref[...], bnx_ref[...])
    xout_ref[...] = (
        jnp.dot(hnew, wo_ref[...], preferred_element_type=jnp.float32)
        + bo_ref[...]
    )


def _h_update_last(h, uh, agg0, agg1, gnx_i, bnx_i, woutx, boutx):
    n, d = h.shape
    m = woutx.shape[1]
    vec = pl.BlockSpec((1, d), lambda i: (0, 0))
    blk = pl.BlockSpec((_BN, d), lambda i: (i, 0))
    return pl.pallas_call(
        _hupd_last_body,
        grid=(n // _BN,),
        in_specs=[blk, blk, blk, blk, vec, vec,
                  pl.BlockSpec((d, m), lambda i: (0, 0)),
                  pl.BlockSpec((1, m), lambda i: (0, 0))],
        out_specs=pl.BlockSpec((_BN, m), lambda i: (i, 0)),
        out_shape=jax.ShapeDtypeStruct((n, m), jnp.float32),
    )(h, uh, agg0, agg1, gnx_i.reshape(1, d), bnx_i.reshape(1, d),
      woutx, boutx.reshape(1, m))


def _hupd_body(h_ref, uh_ref, a0_ref, a1_ref, gnx_ref, bnx_ref, out_ref):
    out_ref[...] = h_ref[...] + _ln_act(
        uh_ref[...] + a0_ref[...] + a1_ref[...], gnx_ref[...], bnx_ref[...])


def _h_update(h, uh, agg0, agg1, gnx_i, bnx_i):
    n, d = h.shape
    vec = pl.BlockSpec((1, d), lambda i: (0, 0))
    blk = pl.BlockSpec((_BN, d), lambda i: (i, 0))
    return pl.pallas_call(
        _hupd_body,
        grid=(n // _BN,),
        in_specs=[blk, blk, blk, blk, vec, vec],
        out_specs=blk,
        out_shape=jax.ShapeDtypeStruct((n, d), jnp.float32),
    )(h, uh, agg0, agg1, gnx_i.reshape(1, d), bnx_i.reshape(1, d))


def _scg_body(bh, ch, src3, dst3, s_out,
              idxs, idxd, bb0, bc0, bb1, bc1,
              semb0, semc0, semb1, semc1, semo0, semo1):
    c = lax.axis_index("c")
    s = lax.axis_index("s")
    wid = s * _NC + c
    chunks = src3.shape[1]
    base = wid * chunks * _KC
    pltpu.sync_copy(src3.at[wid], idxs)
    pltpu.sync_copy(dst3.at[wid], idxd)

    def start(ci, bb, bc, semb, semc):
        pltpu.async_copy(bh.at[idxs.at[ci]], bb, semb)
        pltpu.async_copy(ch.at[idxd.at[ci]], bc, semc)

    def work(ci, bb, bc, semb, semc, semo):
        pltpu.make_async_copy(bh.at[pl.ds(0, _KC)], bb, semb).wait()
        pltpu.make_async_copy(ch.at[pl.ds(0, _KC)], bc, semc).wait()

        def row(r, rc):
            for j in range(8):
                sl = (r, pl.ds(j * 16, 16))
                bb[sl] = bb[sl] + bc[sl]
            return rc

        lax.fori_loop(0, _KC, row, 0)
        pltpu.async_copy(bb, s_out.at[pl.ds(base + ci * _KC, _KC)], semo)

    def wait_store(bb, semo):
        pltpu.make_async_copy(bb, s_out.at[pl.ds(base, _KC)], semo).wait()

    start(0, bb0, bc0, semb0, semc0)

    def step(t, carry):
        @pl.when(t % 2 == 0)
        def _():
            @pl.when(t > 0)
            def _():
                wait_store(bb1, semo1)

            @pl.when(t + 1 < chunks)
            def _():
                start(t + 1, bb1, bc1, semb1, semc1)

            work(t, bb0, bc0, semb0, semc0, semo0)

        @pl.when(t % 2 == 1)
        def _():
            wait_store(bb0, semo0)

            @pl.when(t + 1 < chunks)
            def _():
                start(t + 1, bb0, bc0, semb0, semc0)

            work(t, bb1, bc1, semb1, semc1, semo1)

        return carry

    lax.fori_loop(0, chunks, step, 0)
    wait_store(bb0, semo0)


def _sc_gather_s(bh, ch, src3, dst3):
    ecount = src3.size
    d = bh.shape[1]
    chunks = src3.shape[1]
    assert chunks % 2 == 1
    buf = pltpu.VMEM((_KC, d), jnp.float32)
    idx = pltpu.VMEM((chunks, _KC), jnp.int32)
    return pl.kernel(
        _scg_body,
        out_type=jax.ShapeDtypeStruct((ecount, d), jnp.float32),
        mesh=plsc.VectorSubcoreMesh(core_axis_name="c", subcore_axis_name="s"),
        scratch_types=[idx, idx, buf, buf, buf, buf]
        + [pltpu.SemaphoreType.DMA] * 6,
    )(bh, ch, src3, dst3)


def _scs_body(gates, vh, src, dst, agg0, agg1,
              is0, id0, is1, id1, bg0, bv0, bg1, bv1, acc,
              semg0, semv0, semg1, semv1, sems0, sems1):
    c = lax.axis_index("c")
    s = lax.axis_index("s")
    wid = s * _NC + c
    n = vh.shape[0]
    per_w = gates.shape[0] // _NW
    chunks = per_w // _KC
    base = wid * per_w
    nrch = n // _KC

    zero = jnp.zeros((16,), jnp.float32)

    def zrow(r, carry):
        for j in range(8):
            bv0[r, pl.ds(j * 16, 16)] = zero
        return carry

    lax.fori_loop(0, _KC, zrow, 0)

    def zcopy(t, carry):
        ct = t * _NS + s

        @pl.when(ct < nrch)
        def _():
            pltpu.sync_copy(bv0, acc.at[pl.ds(ct * _KC, _KC)])

        return carry

    lax.fori_loop(0, pl.cdiv(nrch, _NS), zcopy, 0)
    plsc.subcore_barrier()

    def start(ci, isb, idb, bg, bv, semg, semv):
        off = base + ci * _KC
        pltpu.sync_copy(src.at[pl.ds(off, _KC)], isb)
        pltpu.sync_copy(dst.at[pl.ds(off, _KC)], idb)
        pltpu.async_copy(vh.at[isb], bv, semv)
        pltpu.async_copy(gates.at[pl.ds(off, _KC)], bg, semg)

    def work(ci, idb, bg, bv, semg, semv, semsc):
        pltpu.make_async_copy(vh.at[pl.ds(0, _KC)], bv, semv).wait()
        pltpu.make_async_copy(vh.at[pl.ds(0, _KC)], bg, semg).wait()

        def row(r, rc):
            for j in range(8):
                sl = (r, pl.ds(j * 16, 16))
                bv[sl] = bv[sl] * bg[sl]
            return rc

        lax.fori_loop(0, _KC, row, 0)
        pltpu.async_copy(bv, acc.at[idb], semsc, add=True)

    def wait_scat(bv, semsc):
        pltpu.make_async_copy(bv, acc.at[pl.ds(0, _KC)], semsc).wait()

    start(0, is0, id0, bg0, bv0, semg0, semv0)

    def step(t, carry):
        @pl.when(t % 2 == 0)
        def _():
            @pl.when(t > 0)
            def _():
                wait_scat(bv1, sems1)

            @pl.when(t + 1 < chunks)
            def _():
                start(t + 1, is1, id1, bg1, bv1, semg1, semv1)

            work(t, id0, bg0, bv0, semg0, semv0, sems0)

        @pl.when(t % 2 == 1)
        def _():
            wait_scat(bv0, sems0)

            @pl.when(t + 1 < chunks)
            def _():
                start(t + 1, is0, id0, bg0, bv0, semg0, semv0)

            work(t, id1, bg1, bv1, semg1, semv1, sems1)

        return carry

    lax.fori_loop(0, chunks, step, 0)
    wait_scat(bv0, sems0)
    plsc.subcore_barrier()

    def flush(t, carry):
        ct = t * _NS + s

        @pl.when(ct < nrch)
        def _():
            r0 = ct * _KC
            pltpu.sync_copy(acc.at[pl.ds(r0, _KC)], bv0)

            @pl.when(c == 0)
            def _():
                pltpu.sync_copy(bv0, agg0.at[pl.ds(r0, _KC)])

            @pl.when(c == 1)
            def _():
                pltpu.sync_copy(bv0, agg1.at[pl.ds(r0, _KC)])

        return carry

    lax.fori_loop(0, pl.cdiv(nrch, _NS), flush, 0)


def _sc_scatter(gates, vh, src, dst):
    n, d = vh.shape
    assert (gates.shape[0] // _NW // _KC) % 2 == 1
    out = jax.ShapeDtypeStruct((n, d), jnp.float32)
    buf = pltpu.VMEM((_KC, d), jnp.float32)
    idx = pltpu.VMEM((_KC,), jnp.int32)
    return pl.kernel(
        _scs_body,
        out_type=[out, out],
        mesh=plsc.VectorSubcoreMesh(core_axis_name="c", subcore_axis_name="s"),
        scratch_types=[idx, idx, idx, idx, buf, buf, buf, buf,
                       pltpu.VMEM_SHARED((n, d), jnp.float32)]
        + [pltpu.SemaphoreType.DMA] * 6,
    )(gates, vh, src, dst)


def kernel(x, e, edge_index, W_xe, b_xe, W_ee, b_ee, U, Ub, V, Vb, A, Ab,
           Bm, Bb, Cm, Cb, gnx, bnx, gne, bne, W_outx, b_outx, W_oute,
           b_oute):
    d = W_xe.shape[1]
    layers = U.shape[0]
    ecount = e.shape[0]
    chunks = ecount // _NW // _KC
    src1 = edge_index[0]
    dst1 = edge_index[1]
    src3 = src1.reshape(_NW, chunks, _KC)
    dst3 = dst1.reshape(_NW, chunks, _KC)

    h = x @ W_xe + b_xe

    g = None
    e_out = x_out = None
    for i in range(layers):
        wcat = jnp.concatenate([U[i], V[i], Bm[i], Cm[i]], axis=1)
        bcat = jnp.concatenate([Ub[i], Vb[i], Bb[i], Cb[i]], axis=0)
        uh, vh, bh, ch = _mm4(h, wcat, bcat)
        s = _sc_gather_s(bh, ch, src3, dst3)
        if i == 0:
            gates, gnew = _edge_fused_first(
                e, W_ee[0], b_ee, s, A[i], Ab[i], gne[i], bne[i])
        elif i == layers - 1:
            gates, e_out = _edge_fused_last(
                g, s, A[i], Ab[i], gne[i], bne[i], W_oute, b_oute)
            gnew = None
        else:
            gates, gnew = _edge_fused(g, s, A[i], Ab[i], gne[i], bne[i])
        agg0, agg1 = _sc_scatter(gates, vh, src1, dst1)
        if i == layers - 1:
            x_out = _h_update_last(
                h, uh, agg0, agg1, gnx[i], bnx[i], W_outx, b_outx)
        else:
            h = _h_update(h, uh, agg0, agg1, gnx[i], bnx[i])
        g = gnew

    return (x_out, e_out)


# speedup vs baseline: 4.4447x; 1.0250x over previous
import functools

import jax
import jax.numpy as jnp
from jax import lax
from jax.experimental import pallas as pl
from jax.experimental.pallas import tpu as pltpu
from jax.experimental.pallas import tpu_sc as plsc

_BN = 1000
_BE = 2000
_NC = 2
_NS = 16
_NW = _NC * _NS
_KC = 80


def _ln_act(v, gamma, beta):
    mu = jnp.mean(v, axis=-1, keepdims=True)
    var = jnp.mean((v - mu) ** 2, axis=-1, keepdims=True)
    return jax.nn.relu(gamma * (v - mu) * jax.lax.rsqrt(var + 1e-5) + beta)


def _mm_body(h_ref, w_ref, b_ref, out_ref):
    out_ref[...] = (
        jnp.dot(h_ref[...], w_ref[...], preferred_element_type=jnp.float32)
        + b_ref[...]
    )


def _mm(h, w, b, bn):
    n, d = h.shape
    m = w.shape[1]
    return pl.pallas_call(
        _mm_body,
        grid=(n // bn,),
        in_specs=[
            pl.BlockSpec((bn, d), lambda i: (i, 0)),
            pl.BlockSpec((d, m), lambda i: (0, 0)),
            pl.BlockSpec((1, m), lambda i: (0, 0)),
        ],
        out_specs=pl.BlockSpec((bn, m), lambda i: (i, 0)),
        out_shape=jax.ShapeDtypeStruct((n, m), jnp.float32),
    )(h, w, b.reshape(1, m))


def _mm4_body(h_ref, w_ref, b_ref, o0, o1, o2, o3):
    hw = (
        jnp.dot(h_ref[...], w_ref[...], preferred_element_type=jnp.float32)
        + b_ref[...]
    )
    d = o0.shape[1]
    o0[...] = hw[:, :d]
    o1[...] = hw[:, d:2 * d]
    o2[...] = hw[:, 2 * d:3 * d]
    o3[...] = hw[:, 3 * d:]


def _mm4(h, wcat, bcat):
    n, d = h.shape
    blk = pl.BlockSpec((_BN, d), lambda i: (i, 0))
    out = jax.ShapeDtypeStruct((n, d), jnp.float32)
    return pl.pallas_call(
        _mm4_body,
        grid=(n // _BN,),
        in_specs=[
            blk,
            pl.BlockSpec((d, 4 * d), lambda i: (0, 0)),
            pl.BlockSpec((1, 4 * d), lambda i: (0, 0)),
        ],
        out_specs=[blk, blk, blk, blk],
        out_shape=[out, out, out, out],
    )(h, wcat, bcat.reshape(1, 4 * d))


def _edge_body(g_ref, s_ref, a_ref, ab_ref, gne_ref, bne_ref,
               gates_ref, gnew_ref):
    g = g_ref[...]
    e_new = (
        jnp.dot(g, a_ref[...], preferred_element_type=jnp.float32)
        + ab_ref[...]
        + s_ref[...]
    )
    gates_ref[...] = jax.nn.sigmoid(e_new)
    gnew_ref[...] = g + _ln_act(e_new, gne_ref[...], bne_ref[...])


def _edge_fused(g, s, a, ab, gne_i, bne_i):
    ecount, d = g.shape
    vec = pl.BlockSpec((1, d), lambda i: (0, 0))
    blk = pl.BlockSpec((_BE, d), lambda i: (i, 0))
    out = jax.ShapeDtypeStruct((ecount, d), jnp.float32)
    return pl.pallas_call(
        _edge_body,
        grid=(ecount // _BE,),
        in_specs=[blk, blk, pl.BlockSpec((d, d), lambda i: (0, 0)),
                  vec, vec, vec],
        out_specs=[blk, blk],
        out_shape=[out, out],
    )(g, s, a, ab.reshape(1, d), gne_i.reshape(1, d), bne_i.reshape(1, d))


def _edge_first_body(e_ref, wee_ref, bee_ref, s_ref, a_ref, ab_ref,
                     gne_ref, bne_ref, gates_ref, gnew_ref):
    g = e_ref[...] * wee_ref[...] + bee_ref[...]
    e_new = (
        jnp.dot(g, a_ref[...], preferred_element_type=jnp.float32)
        + ab_ref[...]
        + s_ref[...]
    )
    gates_ref[...] = jax.nn.sigmoid(e_new)
    gnew_ref[...] = g + _ln_act(e_new, gne_ref[...], bne_ref[...])


def _edge_fused_first(e, wee, bee, s, a, ab, gne_i, bne_i):
    ecount, d = s.shape
    vec = pl.BlockSpec((1, d), lambda i: (0, 0))
    blk = pl.BlockSpec((_BE, d), lambda i: (i, 0))
    out = jax.ShapeDtypeStruct((ecount, d), jnp.float32)
    return pl.pallas_call(
        _edge_first_body,
        grid=(ecount // _BE,),
        in_specs=[pl.BlockSpec((_BE, 1), lambda i: (i, 0)), vec, vec, blk,
                  pl.BlockSpec((d, d), lambda i: (0, 0)), vec, vec, vec],
        out_specs=[blk, blk],
        out_shape=[out, out],
    )(e, wee.reshape(1, d), bee.reshape(1, d), s, a, ab.reshape(1, d),
      gne_i.reshape(1, d), bne_i.reshape(1, d))


def _edge_last_body(g_ref, s_ref, a_ref, ab_ref, gne_ref, bne_ref,
                    wo_ref, bo_ref, gates_ref, eout_ref):
    g = g_ref[...]
    e_new = (
        jnp.dot(g, a_ref[...], preferred_element_type=jnp.float32)
        + ab_ref[...]
        + s_ref[...]
    )
    gates_ref[...] = jax.nn.sigmoid(e_new)
    gnew = g + _ln_act(e_new, gne_ref[...], bne_ref[...])
    eout_ref[...] = (
        jnp.dot(gnew, wo_ref[...], preferred_element_type=jnp.float32)
        + bo_ref[...]
    ).T


def _edge_fused_last(g, s, a, ab, gne_i, bne_i, woute, boute):
    ecount, d = g.shape
    m = woute.shape[1]
    be = 2560
    vec = pl.BlockSpec((1, d), lambda i: (0, 0))
    blk = pl.BlockSpec((be, d), lambda i: (i, 0))
    return pl.pallas_call(
        _edge_last_body,
        grid=(ecount // be,),
        in_specs=[blk, blk, pl.BlockSpec((d, d), lambda i: (0, 0)),
                  vec, vec, vec,
                  pl.BlockSpec((d, m), lambda i: (0, 0)),
                  pl.BlockSpec((1, m), lambda i: (0, 0))],
        out_specs=[blk, pl.BlockSpec((m, be), lambda i: (0, i))],
        out_shape=[jax.ShapeDtypeStruct((ecount, d), jnp.float32),
                   jax.ShapeDtypeStruct((m, ecount), jnp.float32)],
    )(g, s, a, ab.reshape(1, d), gne_i.reshape(1, d), bne_i.reshape(1, d),
      woute, boute.reshape(1, m))


def _hupd_last_body(h_ref, uh_ref, a0_ref, a1_ref, gnx_ref, bnx_ref,
                    wo_ref, bo_ref, xout_ref):
    hnew = h_ref[...] + _ln_act(
        uh_ref[...] + a0_ref[...] + a1_ref[...], gnx_ref[...], bnx_ref[...])
    xout_ref[...] = (
        jnp.dot(hnew, wo_ref[...], preferred_element_type=jnp.float32)
        + bo_ref[...]
    )


def _h_update_last(h, uh, agg0, agg1, gnx_i, bnx_i, woutx, boutx):
    n, d = h.shape
    m = woutx.shape[1]
    vec = pl.BlockSpec((1, d), lambda i: (0, 0))
    blk = pl.BlockSpec((_BN, d), lambda i: (i, 0))
    return pl.pallas_call(
        _hupd_last_body,
        grid=(n // _BN,),
        in_specs=[blk, blk, blk, blk, vec, vec,
                  pl.BlockSpec((d, m), lambda i: (0, 0)),
                  pl.BlockSpec((1, m), lambda i: (0, 0))],
        out_specs=pl.BlockSpec((_BN, m), lambda i: (i, 0)),
        out_shape=jax.ShapeDtypeStruct((n, m), jnp.float32),
    )(h, uh, agg0, agg1, gnx_i.reshape(1, d), bnx_i.reshape(1, d),
      woutx, boutx.reshape(1, m))


def _hupd_body(h_ref, uh_ref, a0_ref, a1_ref, gnx_ref, bnx_ref, out_ref):
    out_ref[...] = h_ref[...] + _ln_act(
        uh_ref[...] + a0_ref[...] + a1_ref[...], gnx_ref[...], bnx_ref[...])


def _h_update(h, uh, agg0, agg1, gnx_i, bnx_i):
    n, d = h.shape
    vec = pl.BlockSpec((1, d), lambda i: (0, 0))
    blk = pl.BlockSpec((_BN, d), lambda i: (i, 0))
    return pl.pallas_call(
        _hupd_body,
        grid=(n // _BN,),
        in_specs=[blk, blk, blk, blk, vec, vec],
        out_specs=blk,
        out_shape=jax.ShapeDtypeStruct((n, d), jnp.float32),
    )(h, uh, agg0, agg1, gnx_i.reshape(1, d), bnx_i.reshape(1, d))


def _scg_body(bh, ch, src3, dst3, s_out,
              idxs, idxd, bb0, bc0, bb1, bc1,
              semb0, semc0, semb1, semc1, semo0, semo1):
    c = lax.axis_index("c")
    s = lax.axis_index("s")
    wid = s * _NC + c
    chunks = src3.shape[1]
    base = wid * chunks * _KC
    pltpu.sync_copy(src3.at[wid], idxs)
    pltpu.sync_copy(dst3.at[wid], idxd)

    def start(ci, bb, bc, semb, semc):
        pltpu.async_copy(bh.at[idxs.at[ci]], bb, semb)
        pltpu.async_copy(ch.at[idxd.at[ci]], bc, semc)

    def work(ci, bb, bc, semb, semc, semo):
        pltpu.make_async_copy(bh.at[pl.ds(0, _KC)], bb, semb).wait()
        pltpu.make_async_copy(ch.at[pl.ds(0, _KC)], bc, semc).wait()

        def row(r, rc):
            for j in range(8):
                sl = (r, pl.ds(j * 16, 16))
                bb[sl] = bb[sl] + bc[sl]
            return rc

        lax.fori_loop(0, _KC, row, 0)
        pltpu.async_copy(bb, s_out.at[pl.ds(base + ci * _KC, _KC)], semo)

    def wait_store(bb, semo):
        pltpu.make_async_copy(bb, s_out.at[pl.ds(base, _KC)], semo).wait()

    start(0, bb0, bc0, semb0, semc0)

    def step(t, carry):
        @pl.when(t % 2 == 0)
        def _():
            @pl.when(t > 0)
            def _():
                wait_store(bb1, semo1)

            @pl.when(t + 1 < chunks)
            def _():
                start(t + 1, bb1, bc1, semb1, semc1)

            work(t, bb0, bc0, semb0, semc0, semo0)

        @pl.when(t % 2 == 1)
        def _():
            wait_store(bb0, semo0)

            @pl.when(t + 1 < chunks)
            def _():
                start(t + 1, bb0, bc0, semb0, semc0)

            work(t, bb1, bc1, semb1, semc1, semo1)

        return carry

    lax.fori_loop(0, chunks, step, 0)
    wait_store(bb0, semo0)


def _sc_gather_s(bh, ch, src3, dst3):
    ecount = src3.size
    d = bh.shape[1]
    chunks = src3.shape[1]
    assert chunks % 2 == 1
    buf = pltpu.VMEM((_KC, d), jnp.float32)
    idx = pltpu.VMEM((chunks, _KC), jnp.int32)
    return pl.kernel(
        _scg_body,
        out_type=jax.ShapeDtypeStruct((ecount, d), jnp.float32),
        mesh=plsc.VectorSubcoreMesh(core_axis_name="c", subcore_axis_name="s"),
        scratch_types=[idx, idx, buf, buf, buf, buf]
        + [pltpu.SemaphoreType.DMA] * 6,
    )(bh, ch, src3, dst3)


def _scs_body(gates, vh, src, dst, agg0, agg1,
              is0, id0, is1, id1, bg0, bv0, bg1, bv1, acc,
              semg0, semv0, semg1, semv1, sems0, sems1):
    c = lax.axis_index("c")
    s = lax.axis_index("s")
    wid = s * _NC + c
    n = vh.shape[0]
    per_w = gates.shape[0] // _NW
    chunks = per_w // _KC
    base = wid * per_w
    nrch = n // _KC

    zero = jnp.zeros((16,), jnp.float32)

    def zrow(r, carry):
        for j in range(8):
            bv0[r, pl.ds(j * 16, 16)] = zero
        return carry

    lax.fori_loop(0, _KC, zrow, 0)

    def zcopy(t, carry):
        ct = t * _NS + s

        @pl.when(ct < nrch)
        def _():
            pltpu.sync_copy(bv0, acc.at[pl.ds(ct * _KC, _KC)])

        return carry

    lax.fori_loop(0, pl.cdiv(nrch, _NS), zcopy, 0)
    plsc.subcore_barrier()

    def start(ci, isb, idb, bg, bv, semg, semv):
        off = base + ci * _KC
        pltpu.sync_copy(src.at[pl.ds(off, _KC)], isb)
        pltpu.sync_copy(dst.at[pl.ds(off, _KC)], idb)
        pltpu.async_copy(vh.at[isb], bv, semv)
        pltpu.async_copy(gates.at[pl.ds(off, _KC)], bg, semg)

    def work(ci, idb, bg, bv, semg, semv, semsc):
        pltpu.make_async_copy(vh.at[pl.ds(0, _KC)], bv, semv).wait()
        pltpu.make_async_copy(vh.at[pl.ds(0, _KC)], bg, semg).wait()

        def row(r, rc):
            for j in range(8):
                sl = (r, pl.ds(j * 16, 16))
                bv[sl] = bv[sl] * bg[sl]
            return rc

        lax.fori_loop(0, _KC, row, 0)
        pltpu.async_copy(bv, acc.at[idb], semsc, add=True)

    def wait_scat(bv, semsc):
        pltpu.make_async_copy(bv, acc.at[pl.ds(0, _KC)], semsc).wait()

    start(0, is0, id0, bg0, bv0, semg0, semv0)

    def step(t, carry):
        @pl.when(t % 2 == 0)
        def _():
            @pl.when(t > 0)
            def _():
                wait_scat(bv1, sems1)

            @pl.when(t + 1 < chunks)
            def _():
                start(t + 1, is1, id1, bg1, bv1, semg1, semv1)

            work(t, id0, bg0, bv0, semg0, semv0, sems0)

        @pl.when(t % 2 == 1)
        def _():
            wait_scat(bv0, sems0)

            @pl.when(t + 1 < chunks)
            def _():
                start(t + 1, is0, id0, bg0, bv0, semg0, semv0)

            work(t, id1, bg1, bv1, semg1, semv1, sems1)

        return carry

    lax.fori_loop(0, chunks, step, 0)
    wait_scat(bv0, sems0)
    plsc.subcore_barrier()

    def flush(t, carry):
        ct = t * _NS + s

        @pl.when(ct < nrch)
        def _():
            r0 = ct * _KC
            pltpu.sync_copy(acc.at[pl.ds(r0, _KC)], bv0)

            @pl.when(c == 0)
            def _():
                pltpu.sync_copy(bv0, agg0.at[pl.ds(r0, _KC)])

            @pl.when(c == 1)
            def _():
                pltpu.sync_copy(bv0, agg1.at[pl.ds(r0, _KC)])

        return carry

    lax.fori_loop(0, pl.cdiv(nrch, _NS), flush, 0)


def _sc_scatter(gates, vh, src, dst):
    n, d = vh.shape
    assert (gates.shape[0] // _NW // _KC) % 2 == 1
    out = jax.ShapeDtypeStruct((n, d), jnp.float32)
    buf = pltpu.VMEM((_KC, d), jnp.float32)
    idx = pltpu.VMEM((_KC,), jnp.int32)
    return pl.kernel(
        _scs_body,
        out_type=[out, out],
        mesh=plsc.VectorSubcoreMesh(core_axis_name="c", subcore_axis_name="s"),
        scratch_types=[idx, idx, idx, idx, buf, buf, buf, buf,
                       pltpu.VMEM_SHARED((n, d), jnp.float32)]
        + [pltpu.SemaphoreType.DMA] * 6,
    )(gates, vh, src, dst)


def kernel(x, e, edge_index, W_xe, b_xe, W_ee, b_ee, U, Ub, V, Vb, A, Ab,
           Bm, Bb, Cm, Cb, gnx, bnx, gne, bne, W_outx, b_outx, W_oute,
           b_oute):
    d = W_xe.shape[1]
    layers = U.shape[0]
    ecount = e.shape[0]
    chunks = ecount // _NW // _KC
    src1 = edge_index[0]
    dst1 = edge_index[1]
    src3 = src1.reshape(_NW, chunks, _KC)
    dst3 = dst1.reshape(_NW, chunks, _KC)

    h = x @ W_xe + b_xe
    g = e * W_ee[0] + b_ee

    e_out_t = x_out = None
    for i in range(layers):
        wcat = jnp.concatenate([U[i], V[i], Bm[i], Cm[i]], axis=1)
        bcat = jnp.concatenate([Ub[i], Vb[i], Bb[i], Cb[i]], axis=0)
        uh, vh, bh, ch = _mm4(h, wcat, bcat)
        s = _sc_gather_s(bh, ch, src3, dst3)
        if i == layers - 1:
            gates, e_out_t = _edge_fused_last(
                g, s, A[i], Ab[i], gne[i], bne[i], W_oute, b_oute)
            gnew = None
        else:
            gates, gnew = _edge_fused(g, s, A[i], Ab[i], gne[i], bne[i])
        agg0, agg1 = _sc_scatter(gates, vh, src1, dst1)
        if i == layers - 1:
            x_out = _h_update_last(
                h, uh, agg0, agg1, gnx[i], bnx[i], W_outx, b_outx)
        else:
            h = _h_update(h, uh, agg0, agg1, gnx[i], bnx[i])
        g = gnew

    return (x_out, e_out_t.T)
